# baseline (speedup 1.0000x reference)
import functools
import math

import jax
import jax.numpy as jnp
from jax import lax
from jax.experimental import pallas as pl
from jax.experimental.pallas import tpu as pltpu
from jax.experimental.pallas import tpu_sc as plsc

VOCAB = 1000000
D = 64
BATCH = 16384
HIST = 50
SCALE = math.sqrt(D)

NC = 2
NS = 16
NW = NC * NS

G = 128
TOTAL = BATCH * HIST
NGROUPS = TOTAL // G
GROUPS_PER_W = NGROUPS // NW


def _sc_body(table_hbm, idx_hbm, out_hbm, idx_v, rows_v, sem):
    wid = lax.axis_index("s") * NC + lax.axis_index("c")
    g0 = wid * GROUPS_PER_W

    def group(i, _):
        grp = g0 + i
        pltpu.sync_copy(idx_hbm.at[grp], idx_v)
        pltpu.async_copy(table_hbm.at[idx_v], rows_v, sem).wait()

        def scale_row(r, _):
            for j in range(D // 16):
                sl = pl.ds(j * 16, 16)
                rows_v[r, sl] = rows_v[r, sl] * SCALE
            return _

        lax.fori_loop(0, G, scale_row, None)
        pltpu.sync_copy(rows_v, out_hbm.at[grp])
        return _

    lax.fori_loop(0, GROUPS_PER_W, group, None)


@jax.jit
def kernel(x, table):
    idx = x.reshape(NGROUPS, G).astype(jnp.int32)
    mesh = plsc.VectorSubcoreMesh(
        core_axis_name="c", subcore_axis_name="s",
        num_cores=NC, num_subcores=NS)
    run = pl.kernel(
        _sc_body,
        out_type=jax.ShapeDtypeStruct((NGROUPS, G, D), jnp.float32),
        mesh=mesh,
        scratch_types=[
            pltpu.VMEM((G,), jnp.int32),
            pltpu.VMEM((G, D), jnp.float32),
            pltpu.SemaphoreType.DMA,
        ],
        compiler_params=pltpu.CompilerParams(use_tc_tiling_on_sc=False),
    )
    out = run(table, idx)
    return out.reshape(BATCH, HIST, D)


# speedup vs baseline: 1.0063x; 1.0063x over previous
import functools
import math

import jax
import jax.numpy as jnp
from jax import lax
from jax.experimental import pallas as pl
from jax.experimental.pallas import tpu as pltpu
from jax.experimental.pallas import tpu_sc as plsc

VOCAB = 1000000
D = 64
BATCH = 16384
HIST = 50
SCALE = math.sqrt(D)

NC = 2
NS = 16
NW = NC * NS

G = 128
TOTAL = BATCH * HIST
NGROUPS = TOTAL // G
STEPS = NGROUPS // NW


def _sc_body(table_hbm, idx_hbm, out_hbm,
             idx_v, ga, gb, sa, sb,
             gsem_a, gsem_b, ssem_a, ssem_b):
    wid = lax.axis_index("s") * NC + lax.axis_index("c")
    base = wid * STEPS

    pltpu.sync_copy(idx_hbm.at[pl.ds(base, STEPS)], idx_v)

    gbufs = (ga, gb)
    sbufs = (sa, sb)
    gsems = (gsem_a, gsem_b)
    ssems = (ssem_a, ssem_b)

    def gather_start(s, b):
        pltpu.async_copy(table_hbm.at[idx_v.at[s]], gbufs[b], gsems[b])

    gather_start(0, 0)
    gather_start(1, 1)

    def pair(i, _):
        for b in range(2):
            s = 2 * i + b
            pltpu.make_async_copy(
                table_hbm.at[idx_v.at[s]], gbufs[b], gsems[b]).wait()

            @pl.when(i >= 1)
            def _():
                pltpu.make_async_copy(
                    sbufs[b], out_hbm.at[base + s - 2], ssems[b]).wait()

            def row(r, c):
                for j in range(D // 16):
                    sl = pl.ds(16 * j, 16)
                    sbufs[b][r, sl] = gbufs[b][r, sl] * SCALE
                return c

            lax.fori_loop(0, G, row, 0, unroll=2)

            pltpu.async_copy(sbufs[b], out_hbm.at[base + s], ssems[b])

            @pl.when(i < STEPS // 2 - 1)
            def _():
                gather_start(s + 2, b)
        return _

    lax.fori_loop(0, STEPS // 2, pair, None)

    for b in range(2):
        pltpu.make_async_copy(
            sbufs[b], out_hbm.at[base + STEPS - 2 + b], ssems[b]).wait()


@jax.jit
def kernel(x, table):
    idx = x.reshape(NGROUPS, G).astype(jnp.int32)
    mesh = plsc.VectorSubcoreMesh(
        core_axis_name="c", subcore_axis_name="s",
        num_cores=NC, num_subcores=NS)
    run = pl.kernel(
        _sc_body,
        out_type=jax.ShapeDtypeStruct((NGROUPS, G, D), jnp.float32),
        mesh=mesh,
        scratch_types=[
            pltpu.VMEM((STEPS, G), jnp.int32),
            pltpu.VMEM((G, D), jnp.float32),
            pltpu.VMEM((G, D), jnp.float32),
            pltpu.VMEM((G, D), jnp.float32),
            pltpu.VMEM((G, D), jnp.float32),
            pltpu.SemaphoreType.DMA,
            pltpu.SemaphoreType.DMA,
            pltpu.SemaphoreType.DMA,
            pltpu.SemaphoreType.DMA,
        ],
        compiler_params=pltpu.CompilerParams(use_tc_tiling_on_sc=False),
    )
    out = run(table, idx)
    return out.reshape(BATCH, HIST, D)
